# baseline (speedup 1.0000x reference)
import jax
import jax.numpy as jnp
from jax.experimental import pallas as pl
from jax.experimental.pallas import tpu as pltpu

_FIELDS = [
    ("ip1", 256, 8), ("ip2", 256, 8), ("ip3", 256, 8), ("regionid", 35, 6),
    ("cityid", 370, 9), ("adexchange", 9, 4), ("url", 2, 1), ("aurl", 2, 1),
    ("adslotw", 21, 5), ("adsloth", 14, 4), ("adslotv", 7, 3), ("adslotfp", 275, 9),
    ("creativeid", 57, 6), ("bidprice", 2, 1), ("payprice", 295, 9), ("userids", 69, 7),
]
_NF = len(_FIELDS)
_VOCAB = [v for (_, v, _) in _FIELDS]
_V16 = [-(-v // 16) * 16 for v in _VOCAB]
_D = [8 * m for (_, _, m) in _FIELDS]
_OFF = [0]
for _d in _D[:-1]:
    _OFF.append(_OFF[-1] + _d)
_TOTAL_D = _OFF[-1] + _D[-1]

_WINDOWS = []
for _w0 in range(0, _TOTAL_D, 128):
    _w1 = min(_w0 + 128, _TOTAL_D)
    _pieces, _rows = [], 0
    for _f in range(_NF):
        _s, _e = max(_OFF[_f], _w0), min(_OFF[_f] + _D[_f], _w1)
        if _s < _e:
            _pieces.append((_f, _s - _OFF[_f], _e - _OFF[_f], _s - _w0, _rows))
            _rows += _V16[_f]
    _WINDOWS.append((_w1 - _w0, _rows, _pieces))

_B = 16384
_BB = 4096
_GRID = _B // _BB


def _body(*refs):
    idx_refs = refs[:_NF]
    t_refs = refs[_NF:2 * _NF]
    out_ref = refs[2 * _NF]
    raw = refs[2 * _NF + 1:3 * _NF + 1]
    bins = refs[3 * _NF + 1:3 * _NF + 1 + len(_WINDOWS)]
    sem = refs[-1]

    @pl.when(pl.program_id(0) == 0)
    def _():
        for f in range(_NF):
            pltpu.make_async_copy(t_refs[f], raw[f], sem).start()
        for f in range(_NF):
            pltpu.make_async_copy(t_refs[f], raw[f], sem).wait()
        for wi, (width, rows, pieces) in enumerate(_WINDOWS):
            bins[wi][...] = jnp.zeros((rows, width), jnp.bfloat16)
            for (f, cs, ce, cw, rb) in pieces:
                bins[wi][rb:rb + _VOCAB[f], cw:cw + (ce - cs)] = (
                    raw[f][:, cs:ce].astype(jnp.bfloat16))

    oh_cache = {}

    def field_oh(f):
        if f not in oh_cache:
            idx = idx_refs[f][...]
            iota = jax.lax.broadcasted_iota(jnp.int32, (_V16[f], _BB), 0)
            oh_cache[f] = (iota == idx[None, :]).astype(jnp.bfloat16)
        return oh_cache[f]

    col = 0
    for wi, (width, rows, pieces) in enumerate(_WINDOWS):
        ohs = [field_oh(f) for (f, _cs, _ce, _cw, _rb) in pieces]
        oh = ohs[0] if len(ohs) == 1 else jnp.concatenate(ohs, axis=0)
        res_t = jax.lax.dot_general(
            bins[wi][...], oh, (((0,), (0,)), ((), ())),
            preferred_element_type=jnp.float32)
        out_ref[:, col:col + width] = res_t.T
        col += width


@jax.jit
def _onehot_embed(idxs, tables):
    return pl.pallas_call(
        _body,
        grid=(_GRID,),
        in_specs=[pl.BlockSpec((_BB,), lambda i: (i,)) for _ in range(_NF)]
        + [pl.BlockSpec(memory_space=pltpu.HBM) for _ in range(_NF)],
        out_specs=pl.BlockSpec((_BB, _TOTAL_D), lambda i: (i, 0)),
        out_shape=jax.ShapeDtypeStruct((_B, _TOTAL_D), jnp.float32),
        scratch_shapes=[pltpu.VMEM((v, d), jnp.float32)
                        for (v, d) in zip(_VOCAB, _D)]
        + [pltpu.VMEM((r, w), jnp.bfloat16) for (w, r, _) in _WINDOWS]
        + [pltpu.SemaphoreType.DMA],
    )(*idxs, *tables)


def kernel(ip1_idx, W_ip1, ip2_idx, W_ip2, ip3_idx, W_ip3, regionid_idx, W_regionid, cityid_idx, W_cityid, adexchange_idx, W_adexchange, url_idx, W_url, aurl_idx, W_aurl, adslotw_idx, W_adslotw, adsloth_idx, W_adsloth, adslotv_idx, W_adslotv, adslotfp_idx, W_adslotfp, creativeid_idx, W_creativeid, bidprice_idx, W_bidprice, payprice_idx, W_payprice, userids_idx, W_userids):
    inp = dict(locals())
    idxs = [inp[name + "_idx"].astype(jnp.int32) for (name, _, _) in _FIELDS]
    tables = [inp["W_" + name] for (name, _, _) in _FIELDS]
    return _onehot_embed(idxs, tables)


# speedup vs baseline: 1.0064x; 1.0064x over previous
import jax
import jax.numpy as jnp
from jax.experimental import pallas as pl
from jax.experimental.pallas import tpu as pltpu

_FIELDS = [
    ("ip1", 256, 8), ("ip2", 256, 8), ("ip3", 256, 8), ("regionid", 35, 6),
    ("cityid", 370, 9), ("adexchange", 9, 4), ("url", 2, 1), ("aurl", 2, 1),
    ("adslotw", 21, 5), ("adsloth", 14, 4), ("adslotv", 7, 3), ("adslotfp", 275, 9),
    ("creativeid", 57, 6), ("bidprice", 2, 1), ("payprice", 295, 9), ("userids", 69, 7),
]
_NF = len(_FIELDS)
_VOCAB = [v for (_, v, _) in _FIELDS]
_V16 = [-(-v // 16) * 16 for v in _VOCAB]
_D = [8 * m for (_, _, m) in _FIELDS]
_OFF = [0]
for _d in _D[:-1]:
    _OFF.append(_OFF[-1] + _d)
_TOTAL_D = _OFF[-1] + _D[-1]

_WINDOWS = []
for _w0 in range(0, _TOTAL_D, 128):
    _w1 = min(_w0 + 128, _TOTAL_D)
    _pieces, _rows = [], 0
    for _f in range(_NF):
        _s, _e = max(_OFF[_f], _w0), min(_OFF[_f] + _D[_f], _w1)
        if _s < _e:
            _pieces.append((_f, _s - _OFF[_f], _e - _OFF[_f], _s - _w0, _rows))
            _rows += _V16[_f]
    _WINDOWS.append((_w1 - _w0, _rows, _pieces))

_B = 16384
_BB = 2048
_GRID = _B // _BB


def _body(*refs):
    idx_refs = refs[:_NF]
    t_refs = refs[_NF:2 * _NF]
    out_ref = refs[2 * _NF]
    raw = refs[2 * _NF + 1:3 * _NF + 1]
    bins = refs[3 * _NF + 1:3 * _NF + 1 + len(_WINDOWS)]
    sem = refs[-1]

    @pl.when(pl.program_id(0) == 0)
    def _():
        for f in range(_NF):
            pltpu.make_async_copy(t_refs[f], raw[f], sem).start()
        for f in range(_NF):
            pltpu.make_async_copy(t_refs[f], raw[f], sem).wait()
        for wi, (width, rows, pieces) in enumerate(_WINDOWS):
            bins[wi][...] = jnp.zeros((rows, width), jnp.bfloat16)
            for (f, cs, ce, cw, rb) in pieces:
                bins[wi][rb:rb + _VOCAB[f], cw:cw + (ce - cs)] = (
                    raw[f][:, cs:ce].astype(jnp.bfloat16))

    oh_cache = {}

    def field_oh(f):
        if f not in oh_cache:
            idx = idx_refs[f][...]
            iota = jax.lax.broadcasted_iota(jnp.int32, (_V16[f], _BB), 0)
            oh_cache[f] = (iota == idx[None, :]).astype(jnp.bfloat16)
        return oh_cache[f]

    col = 0
    for wi, (width, rows, pieces) in enumerate(_WINDOWS):
        ohs = [field_oh(f) for (f, _cs, _ce, _cw, _rb) in pieces]
        oh = ohs[0] if len(ohs) == 1 else jnp.concatenate(ohs, axis=0)
        res_t = jax.lax.dot_general(
            bins[wi][...], oh, (((0,), (0,)), ((), ())),
            preferred_element_type=jnp.float32)
        out_ref[:, col:col + width] = res_t.T
        col += width


@jax.jit
def _onehot_embed(idxs, tables):
    return pl.pallas_call(
        _body,
        grid=(_GRID,),
        in_specs=[pl.BlockSpec((_BB,), lambda i: (i,)) for _ in range(_NF)]
        + [pl.BlockSpec(memory_space=pltpu.HBM) for _ in range(_NF)],
        out_specs=pl.BlockSpec((_BB, _TOTAL_D), lambda i: (i, 0)),
        out_shape=jax.ShapeDtypeStruct((_B, _TOTAL_D), jnp.float32),
        scratch_shapes=[pltpu.VMEM((v, d), jnp.float32)
                        for (v, d) in zip(_VOCAB, _D)]
        + [pltpu.VMEM((r, w), jnp.bfloat16) for (w, r, _) in _WINDOWS]
        + [pltpu.SemaphoreType.DMA],
    )(*idxs, *tables)


def kernel(ip1_idx, W_ip1, ip2_idx, W_ip2, ip3_idx, W_ip3, regionid_idx, W_regionid, cityid_idx, W_cityid, adexchange_idx, W_adexchange, url_idx, W_url, aurl_idx, W_aurl, adslotw_idx, W_adslotw, adsloth_idx, W_adsloth, adslotv_idx, W_adslotv, adslotfp_idx, W_adslotfp, creativeid_idx, W_creativeid, bidprice_idx, W_bidprice, payprice_idx, W_payprice, userids_idx, W_userids):
    inp = dict(locals())
    idxs = [inp[name + "_idx"].astype(jnp.int32) for (name, _, _) in _FIELDS]
    tables = [inp["W_" + name] for (name, _, _) in _FIELDS]
    return _onehot_embed(idxs, tables)
